# Initial kernel scaffold; baseline (speedup 1.0000x reference)
import jax
import jax.numpy as jnp
from jax.experimental import pallas as pl


def kernel(feature, edge_list, W1, as1, ad1, b1, W2, as2, ad2, b2, Wl, bl, Wp, bp):
    raise NotImplementedError("write your pallas kernel here")


# speedup vs baseline: 68.4517x; 68.4517x over previous
import functools

import jax
import jax.numpy as jnp
from jax import lax
from jax.experimental import pallas as pl
from jax.experimental.pallas import tpu as pltpu
from jax.experimental.pallas import tpu_sc as plsc

N = 14
NEG_SLOPE = 0.2
NC, NS = 2, 16
NW = NC * NS


def _sc_counts(src, dst):
    B, E = src.shape
    GW = B // NW
    GC = 8
    n_chunks = GW // GC

    mesh = plsc.VectorSubcoreMesh(core_axis_name="c", subcore_axis_name="s")

    @functools.partial(
        pl.kernel,
        out_type=jax.ShapeDtypeStruct((B, 256), jnp.float32),
        mesh=mesh,
        compiler_params=pltpu.CompilerParams(needs_layout_passes=False),
        scratch_types=[
            pltpu.VMEM((GC, E), jnp.int32),
            pltpu.VMEM((GC, E), jnp.int32),
            pltpu.VMEM((GC * 256,), jnp.float32),
            pltpu.SemaphoreType.DMA,
            pltpu.SemaphoreType.DMA,
            pltpu.SemaphoreType.DMA,
        ],
    )
    def counts_kernel(src_hbm, dst_hbm, out_hbm, src_v, dst_v, hist_v,
                      sem_s, sem_d, sem_out):
        wid = lax.axis_index("s") * NC + lax.axis_index("c")
        base = wid * GW
        zeros16 = jnp.zeros((16,), jnp.float32)
        ones16 = jnp.ones((16,), jnp.float32)

        def chunk_body(ci, carry):
            g0 = base + ci * GC
            cs = pltpu.async_copy(src_hbm.at[pl.ds(g0, GC)], src_v, sem_s)
            cd = pltpu.async_copy(dst_hbm.at[pl.ds(g0, GC)], dst_v, sem_d)
            for k in range(GC * 16):
                hist_v[pl.ds(k * 16, 16)] = zeros16
            cs.wait()
            cd.wait()
            for g in range(GC):
                for c in range(E // 16):
                    s = src_v[g, pl.ds(c * 16, 16)]
                    d = dst_v[g, pl.ds(c * 16, 16)]
                    plsc.addupdate_scatter(
                        hist_v, [g * 256 + s * 16 + d], ones16, mask=s != d
                    )
            for g in range(GC):
                pltpu.async_copy(
                    hist_v.at[pl.ds(g * 256, 256)], out_hbm.at[g0 + g], sem_out
                ).wait()
            return carry

        lax.fori_loop(0, n_chunks, chunk_body, 0)

    return counts_kernel(src, dst)


def _tc_body(
    feat_ref, m_ref, w1_ref, a1_ref, b1_ref, w2_ref, a2_ref, b2_ref,
    wl_ref, bl_ref, wp_ref, bp_ref, pred_ref, a_ref,
):
    bB = feat_ref.shape[0]
    F = feat_ref.shape[2]
    hi = lax.Precision.HIGHEST

    rows = lax.broadcasted_iota(jnp.int32, (16, 16), 0)
    cols = lax.broadcasted_iota(jnp.int32, (16, 16), 1)
    eye = jnp.where((rows == cols) & (rows < N), 1.0, 0.0).astype(jnp.float32)
    M = m_ref[...].reshape(bB, 16, 16) + eye[None, :, :]
    Mv = M[:, :N, :N]
    posv = Mv > 0.0

    def gat_layer(xin, w_ref, a_ref_, b_ref_):
        h = jnp.dot(xin, w_ref[...], preferred_element_type=jnp.float32,
                    precision=hi)
        av = jnp.dot(h, a_ref_[...], preferred_element_type=jnp.float32,
                     precision=hi)
        als = av[:, 0].reshape(bB, N, 1)
        ald = av[:, 1].reshape(bB, 1, N)
        L = als + ald
        L = jnp.where(L >= 0.0, L, NEG_SLOPE * L)
        m = jnp.max(jnp.where(posv, L, -1e30), axis=1, keepdims=True)
        e = jnp.exp(jnp.minimum(L - m, 0.0))
        w = Mv * e
        denom = jnp.maximum(jnp.sum(w, axis=1, keepdims=True), 1e-30)
        P = w / denom
        alpha = jnp.where(posv, e / denom, 0.0)
        h3 = h.reshape(bB, N, F)
        out = lax.dot_general(
            P, h3, (((1,), (1,)), ((0,), (0,))),
            preferred_element_type=jnp.float32, precision=hi,
        )
        out = out.reshape(bB * N, F) + b_ref_[...]
        return out, alpha

    x = feat_ref[...].reshape(bB * N, F)
    o1, _ = gat_layer(x, w1_ref, a1_ref, b1_ref)
    x1 = jnp.maximum(o1, 0.0)
    o2, alpha2 = gat_layer(x1, w2_ref, a2_ref, b2_ref)
    a_ref[...] = alpha2
    v = o2.reshape(bB, N * F)
    hid = jnp.dot(v, wl_ref[...], preferred_element_type=jnp.float32,
                  precision=hi) + bl_ref[...]
    z = jnp.dot(hid, wp_ref[...], preferred_element_type=jnp.float32,
                precision=hi) + bp_ref[...]
    pred_ref[...] = 1.0 / (1.0 + jnp.exp(-z))


def _tc_forward(feature, M256, W1, a1p, b1, W2, a2p, b2, Wl, bl, Wp, bp,
                interpret=False):
    B, _, F = feature.shape
    bB = 128
    grid = (B // bB,)

    def blk(i):
        return (i, 0, 0)

    def blk2(i):
        return (i, 0)

    def full2(i):
        return (0, 0)

    out_shapes = (
        jax.ShapeDtypeStruct((B, 1), jnp.float32),
        jax.ShapeDtypeStruct((B, N, N), jnp.float32),
    )
    return pl.pallas_call(
        _tc_body,
        grid=grid,
        in_specs=[
            pl.BlockSpec((bB, N, F), blk),
            pl.BlockSpec((bB, 256), blk2),
            pl.BlockSpec(W1.shape, full2),
            pl.BlockSpec(a1p.shape, full2),
            pl.BlockSpec(b1.shape, full2),
            pl.BlockSpec(W2.shape, full2),
            pl.BlockSpec(a2p.shape, full2),
            pl.BlockSpec(b2.shape, full2),
            pl.BlockSpec(Wl.shape, full2),
            pl.BlockSpec(bl.shape, full2),
            pl.BlockSpec(Wp.shape, full2),
            pl.BlockSpec(bp.shape, full2),
        ],
        out_specs=(
            pl.BlockSpec((bB, 1), blk2),
            pl.BlockSpec((bB, N, N), blk),
        ),
        out_shape=out_shapes,
        interpret=interpret,
    )(feature, M256, W1, a1p, b1, W2, a2p, b2, Wl, bl, Wp, bp)


def kernel(feature, edge_list, W1, as1, ad1, b1, W2, as2, ad2, b2, Wl, bl, Wp, bp):
    B, _, _ = feature.shape
    src = edge_list[:, :, 0]
    dst = edge_list[:, :, 1]
    M256 = _sc_counts(src, dst)
    a1p = jnp.stack([as1, ad1], axis=1)
    a2p = jnp.stack([as2, ad2], axis=1)
    return _tc_forward(
        feature, M256, W1, a1p, b1.reshape(1, -1), W2, a2p, b2.reshape(1, -1),
        Wl, bl.reshape(1, -1), Wp, bp.reshape(1, -1),
    )
